# baseline (speedup 1.0000x reference)
import functools

import jax
import jax.numpy as jnp
from jax import lax
from jax.experimental import pallas as pl
from jax.experimental.pallas import tpu as pltpu
from jax.experimental.pallas import tpu_sc as plsc

D = 64
NC, NS = 2, 16
NW = NC * NS


def _make_gather(b: int, l: int):
    b_blk = b // NW
    LC = 5
    n_chunks = l // LC

    mesh = plsc.VectorSubcoreMesh(core_axis_name="c", subcore_axis_name="s")

    @functools.partial(
        pl.kernel,
        mesh=mesh,
        compiler_params=pltpu.CompilerParams(use_tc_tiling_on_sc=False),
        out_type=jax.ShapeDtypeStruct((l, b, 2 * D), jnp.float32),
        scratch_types=[
            pltpu.VMEM((l, b_blk), jnp.int32),
            pltpu.VMEM((LC, b_blk, D), jnp.float32),
            pltpu.VMEM((LC, b_blk, D), jnp.float32),
            pltpu.SemaphoreType.DMA,
            pltpu.SemaphoreType.DMA,
            pltpu.SemaphoreType.DMA,
            pltpu.SemaphoreType.DMA,
        ],
    )
    def k(idx_hbm, table_hbm, out_hbm, idx_v, b0, b1, g0, g1, w0, w1):
        wid = lax.axis_index("s") * NC + lax.axis_index("c")
        wb = wid * b_blk
        bufs, gsems, wsems = (b0, b1), (g0, g1), (w0, w1)
        pltpu.sync_copy(idx_hbm.at[pl.ds(0, l), pl.ds(wb, b_blk)], idx_v)

        gload = [None, None]
        write = [None, None]

        def fire(c):
            bb = c % 2
            gload[bb] = [pltpu.async_copy(
                table_hbm.at[idx_v.at[c * LC + j]],
                bufs[bb].at[j],
                gsems[bb],
            ) for j in range(LC)]

        fire(0)
        for c in range(n_chunks):
            bb = c % 2
            if c + 1 < n_chunks:
                nb = (c + 1) % 2
                if write[nb] is not None:
                    write[nb].wait()
                fire(c + 1)
            for h in gload[bb]:
                h.wait()
            write[bb] = pltpu.async_copy(
                bufs[bb],
                out_hbm.at[pl.ds(c * LC, LC), pl.ds(wb, b_blk), pl.ds(0, D)],
                wsems[bb])
        for w in write:
            if w is not None:
                w.wait()

    return k


def _concat_body(gath_ref, other_ref, out_ref):
    l, bm, _ = out_ref.shape
    out_ref[:, :, :D] = gath_ref[:, :, :D]
    out_ref[:, :, D:] = jnp.swapaxes(other_ref[...], 1, 2)


def _make_concat(b: int, l: int, bm: int):
    return pl.pallas_call(
        _concat_body,
        grid=(b // bm,),
        in_specs=[
            pl.BlockSpec((l, bm, 2 * D), lambda i: (0, i, 0)),
            pl.BlockSpec((l, D, bm), lambda i: (0, 0, i)),
        ],
        out_specs=pl.BlockSpec((l, bm, 2 * D), lambda i: (0, i, 0)),
        out_shape=jax.ShapeDtypeStruct((l, b, 2 * D), jnp.float32),
    )


def kernel(indices, other_features, table):
    b, l = indices.shape
    idx_lm = indices.transpose(1, 0).astype(jnp.int32)
    gath3 = _make_gather(b, l)(idx_lm, table)
    other_t = other_features.transpose(1, 2, 0)
    out_t = _make_concat(b, l, 256)(gath3, other_t)
    return out_t.transpose(1, 0, 2)
